# baseline (speedup 1.0000x reference)
import functools

import jax
import jax.numpy as jnp
from jax import lax
from jax.experimental import pallas as pl
from jax.experimental.pallas import tpu as pltpu
from jax.experimental.pallas import tpu_sc as plsc

B, S, D = 4, 2048, 1024
VISION_START = 1536
NC, NS, L = 2, 16, 16
NW = NC * NS
SW = S // NW
NV = D // L
CHS = 8
NSET = SW // CHS


def _rsqrt_vec(v):
    i = plsc.bitcast(v, jnp.int32)
    i = jnp.int32(0x5F3759DF) - (i >> 1)
    y = plsc.bitcast(i, jnp.float32)
    for _ in range(2):
        y = y * (1.5 - 0.5 * v * y * y)
    return y


_mesh = plsc.VectorSubcoreMesh(
    core_axis_name="c", subcore_axis_name="s", num_cores=NC, num_subcores=NS
)


@functools.partial(
    pl.kernel,
    out_type=jax.ShapeDtypeStruct((B, S, D), jnp.float32),
    mesh=_mesh,
    compiler_params=pltpu.CompilerParams(needs_layout_passes=False),
    scratch_types=[
        pltpu.VMEM((2, CHS, D), jnp.float32),
        pltpu.VMEM((2, B, CHS, D), jnp.float32),
        pltpu.VMEM((1, D), jnp.float32),
        pltpu.VMEM((D,), jnp.float32),
        pltpu.VMEM((D,), jnp.float32),
        pltpu.SemaphoreType.DMA,
        pltpu.SemaphoreType.DMA,
        pltpu.SemaphoreType.DMA,
        pltpu.SemaphoreType.DMA,
        pltpu.SemaphoreType.DMA,
    ],
)
def _sc_embed_ln(
    emb, pos, mod, gam, bet, out, pos_v, e_v, mod_v, g_v, b_v, si0, si1, so0, so1, sp
):
    wid = lax.axis_index("s") * NC + lax.axis_index("c")
    s0 = wid * SW

    mrow = jnp.where(s0 >= VISION_START, 1, 0)
    h_mod = pltpu.async_copy(mod.at[pl.ds(mrow, 1)], mod_v, sp)
    h_gam = pltpu.async_copy(gam, g_v, sp)
    h_bet = pltpu.async_copy(bet, b_v, sp)

    in_sems = (si0, si1)
    out_sems = (so0, so1)
    inv_d = 1.0 / D
    zero = jnp.zeros((L,), jnp.float32)

    def start_in(si, k):
        tok0 = s0 + si * CHS
        hs = [pltpu.async_copy(pos.at[pl.ds(tok0, CHS)], pos_v.at[k], in_sems[k])]
        for b in range(B):
            hs.append(
                pltpu.async_copy(
                    emb.at[b, pl.ds(tok0, CHS)], e_v.at[k, b], in_sems[k]
                )
            )
        return hs

    def start_out(si, k):
        tok0 = s0 + si * CHS
        return [
            pltpu.async_copy(e_v.at[k, b], out.at[b, pl.ds(tok0, CHS)], out_sems[k])
            for b in range(B)
        ]

    def compute_set(k):
        @plsc.parallel_loop(0, CHS, step=4)
        def per_pos(sl0):
            mean_vecs = []
            scale_vecs = []
            for d in range(4):
                sl = sl0 + d

                @plsc.parallel_loop(0, NV, unroll=4, carry=(zero,) * 8)
                def p1(j, carry, sl=sl):
                    a0, q0, a1, q1, a2, q2, a3, q3 = carry
                    col = pl.ds(j * L, L)
                    c = pos_v[k, sl, col] + mod_v[0, col]
                    x0 = e_v[k, 0, sl, col] + c
                    e_v[k, 0, sl, col] = x0
                    x1 = e_v[k, 1, sl, col] + c
                    e_v[k, 1, sl, col] = x1
                    x2 = e_v[k, 2, sl, col] + c
                    e_v[k, 2, sl, col] = x2
                    x3 = e_v[k, 3, sl, col] + c
                    e_v[k, 3, sl, col] = x3
                    return (
                        a0 + x0, q0 + x0 * x0,
                        a1 + x1, q1 + x1 * x1,
                        a2 + x2, q2 + x2 * x2,
                        a3 + x3, q3 + x3 * x3,
                    )

                carry = p1
                for b in range(B):
                    s1 = jnp.sum(carry[2 * b])
                    s2 = jnp.sum(carry[2 * b + 1])
                    mean = s1 * inv_d
                    var = s2 * inv_d - mean * mean
                    mean_vecs.append(jnp.full((L,), mean, jnp.float32))
                    scale_vecs.append(
                        _rsqrt_vec(jnp.full((L,), var + 1e-12, jnp.float32))
                    )

            @plsc.parallel_loop(0, NV)
            def p2(j):
                col = pl.ds(j * L, L)
                g = g_v[col]
                bb = b_v[col]
                for d in range(4):
                    sl = sl0 + d
                    for b in range(B):
                        i = 4 * d + b
                        x = e_v[k, b, sl, col]
                        e_v[k, b, sl, col] = (x - mean_vecs[i]) * scale_vecs[i] * g + bb

    def wait_in(t, k):
        tok0 = s0 + t * CHS
        pltpu.make_async_copy(
            pos.at[pl.ds(tok0, CHS)], pos_v.at[k], in_sems[k]
        ).wait()
        for b in range(B):
            pltpu.make_async_copy(
                emb.at[b, pl.ds(tok0, CHS)], e_v.at[k, b], in_sems[k]
            ).wait()

    def wait_out(t, k):
        tok0 = s0 + t * CHS
        for b in range(B):
            pltpu.make_async_copy(
                e_v.at[k, b], out.at[b, pl.ds(tok0, CHS)], out_sems[k]
            ).wait()

    start_in(0, 0)
    h_mod.wait()
    h_gam.wait()
    h_bet.wait()

    @pl.loop(0, NSET, step=2)
    def _ring(si):
        for k in (0, 1):
            t = si + k

            @pl.when(t + 1 < NSET)
            def _():
                @pl.when(t - 1 >= 0)
                def _():
                    wait_out(t - 1, 1 - k)

                start_in(t + 1, 1 - k)

            wait_in(t, k)
            compute_set(k)
            start_out(t, k)

    wait_out(NSET - 2, 0)
    wait_out(NSET - 1, 1)


def kernel(embeddings, position_table, modality_table, ln_gamma, ln_beta):
    return _sc_embed_ln(
        embeddings, position_table, modality_table, ln_gamma, ln_beta
    )


# speedup vs baseline: 1.0503x; 1.0503x over previous
import functools

import jax
import jax.numpy as jnp
from jax import lax
from jax.experimental import pallas as pl
from jax.experimental.pallas import tpu as pltpu
from jax.experimental.pallas import tpu_sc as plsc

B, S, D = 4, 2048, 1024
VISION_START = 1536
NC, NS, L = 2, 16, 16
NW = NC * NS
SW = S // NW
NV = D // L
CHS = 8
NSET = SW // CHS


def _rsqrt_vec(v):
    i = plsc.bitcast(v, jnp.int32)
    i = jnp.int32(0x5F3759DF) - (i >> 1)
    y = plsc.bitcast(i, jnp.float32)
    for _ in range(2):
        y = y * (1.5 - 0.5 * v * y * y)
    return y


_mesh = plsc.VectorSubcoreMesh(
    core_axis_name="c", subcore_axis_name="s", num_cores=NC, num_subcores=NS
)


@functools.partial(
    pl.kernel,
    out_type=jax.ShapeDtypeStruct((B, S, D), jnp.float32),
    mesh=_mesh,
    compiler_params=pltpu.CompilerParams(needs_layout_passes=False),
    scratch_types=[
        pltpu.VMEM((2, CHS, D), jnp.float32),
        pltpu.VMEM((2, B, CHS, D), jnp.float32),
        pltpu.VMEM((1, D), jnp.float32),
        pltpu.VMEM((D,), jnp.float32),
        pltpu.VMEM((D,), jnp.float32),
        pltpu.SemaphoreType.DMA,
        pltpu.SemaphoreType.DMA,
        pltpu.SemaphoreType.DMA,
        pltpu.SemaphoreType.DMA,
        pltpu.SemaphoreType.DMA,
    ],
)
def _sc_embed_ln(
    emb, pos, mod, gam, bet, out, pos_v, e_v, mod_v, g_v, b_v, si0, si1, so0, so1, sp
):
    wid = lax.axis_index("s") * NC + lax.axis_index("c")
    s0 = wid * SW

    mrow = jnp.where(s0 >= VISION_START, 1, 0)
    h_mod = pltpu.async_copy(mod.at[pl.ds(mrow, 1)], mod_v, sp)
    h_gam = pltpu.async_copy(gam, g_v, sp)
    h_bet = pltpu.async_copy(bet, b_v, sp)

    in_sems = (si0, si1)
    out_sems = (so0, so1)
    inv_d = 1.0 / D
    zero = jnp.zeros((L,), jnp.float32)

    def start_in(si, k):
        tok0 = s0 + si * CHS
        hs = [pltpu.async_copy(pos.at[pl.ds(tok0, CHS)], pos_v.at[k], in_sems[k])]
        for b in range(B):
            hs.append(
                pltpu.async_copy(
                    emb.at[b, pl.ds(tok0, CHS)], e_v.at[k, b], in_sems[k]
                )
            )
        return hs

    def start_out(si, k):
        tok0 = s0 + si * CHS
        return [
            pltpu.async_copy(e_v.at[k, b], out.at[b, pl.ds(tok0, CHS)], out_sems[k])
            for b in range(B)
        ]

    def compute_set(k):
        @plsc.parallel_loop(0, CHS, step=2)
        def per_pos(sl0):
            mean_vecs = []
            scale_vecs = []
            for d in range(2):
                sl = sl0 + d

                @plsc.parallel_loop(0, NV, unroll=4, carry=(zero,) * 8)
                def p1(j, carry, sl=sl):
                    a0, q0, a1, q1, a2, q2, a3, q3 = carry
                    col = pl.ds(j * L, L)
                    c = pos_v[k, sl, col] + mod_v[0, col]
                    x0 = e_v[k, 0, sl, col] + c
                    e_v[k, 0, sl, col] = x0
                    x1 = e_v[k, 1, sl, col] + c
                    e_v[k, 1, sl, col] = x1
                    x2 = e_v[k, 2, sl, col] + c
                    e_v[k, 2, sl, col] = x2
                    x3 = e_v[k, 3, sl, col] + c
                    e_v[k, 3, sl, col] = x3
                    return (
                        a0 + x0, q0 + x0 * x0,
                        a1 + x1, q1 + x1 * x1,
                        a2 + x2, q2 + x2 * x2,
                        a3 + x3, q3 + x3 * x3,
                    )

                carry = p1
                for b in range(B):
                    s1 = jnp.sum(carry[2 * b])
                    s2 = jnp.sum(carry[2 * b + 1])
                    mean = s1 * inv_d
                    var = s2 * inv_d - mean * mean
                    mean_vecs.append(jnp.full((L,), mean, jnp.float32))
                    scale_vecs.append(
                        _rsqrt_vec(jnp.full((L,), var + 1e-12, jnp.float32))
                    )

            @plsc.parallel_loop(0, NV, unroll=2)
            def p2(j):
                col = pl.ds(j * L, L)
                g = g_v[col]
                bb = b_v[col]
                for d in range(2):
                    sl = sl0 + d
                    for b in range(B):
                        i = 4 * d + b
                        x = e_v[k, b, sl, col]
                        e_v[k, b, sl, col] = (x - mean_vecs[i]) * scale_vecs[i] * g + bb

    def wait_in(t, k):
        tok0 = s0 + t * CHS
        pltpu.make_async_copy(
            pos.at[pl.ds(tok0, CHS)], pos_v.at[k], in_sems[k]
        ).wait()
        for b in range(B):
            pltpu.make_async_copy(
                emb.at[b, pl.ds(tok0, CHS)], e_v.at[k, b], in_sems[k]
            ).wait()

    def wait_out(t, k):
        tok0 = s0 + t * CHS
        for b in range(B):
            pltpu.make_async_copy(
                e_v.at[k, b], out.at[b, pl.ds(tok0, CHS)], out_sems[k]
            ).wait()

    start_in(0, 0)
    h_mod.wait()
    h_gam.wait()
    h_bet.wait()

    @pl.loop(0, NSET, step=2)
    def _ring(si):
        for k in (0, 1):
            t = si + k

            @pl.when(t + 1 < NSET)
            def _():
                @pl.when(t - 1 >= 0)
                def _():
                    wait_out(t - 1, 1 - k)

                start_in(t + 1, 1 - k)

            wait_in(t, k)
            compute_set(k)
            start_out(t, k)

    wait_out(NSET - 2, 0)
    wait_out(NSET - 1, 1)


def kernel(embeddings, position_table, modality_table, ln_gamma, ln_beta):
    return _sc_embed_ln(
        embeddings, position_table, modality_table, ln_gamma, ln_beta
    )
